# Initial kernel scaffold; baseline (speedup 1.0000x reference)
import jax
import jax.numpy as jnp
from jax.experimental import pallas as pl


def kernel(x_bd):
    raise NotImplementedError("write your pallas kernel here")


# speedup vs baseline: 2.9607x; 2.9607x over previous
import jax
import jax.numpy as jnp
from jax.experimental import pallas as pl
from jax.experimental.pallas import tpu as pltpu

_MARK_AX = 0
_OP_MUL = 1
_ALU_LO = 16
_ALU_HI = 32
_AX_CARRY_LO = 48
_AX_CARRY_HI = 64
_OUTPUT_LO = 80
_OUTPUT_HI = 96
_DIM = 128


def _argmax16(v):
    r, w = v.shape
    iota = jax.lax.broadcasted_iota(jnp.int32, (r, w), 1)
    mx = jnp.max(v, axis=1, keepdims=True)
    return jnp.min(jnp.where(v == mx, iota, w), axis=1, keepdims=True)


def _body(x_ref, o_ref):
    x = x_ref[...]
    r = x.shape[0]
    lane = jax.lax.broadcasted_iota(jnp.int32, (r, _DIM), 1)

    mask = (x[:, _MARK_AX:_MARK_AX + 1] >= 0.5) & (
        x[:, _OP_MUL:_OP_MUL + 1] >= 0.5)

    a_lo = _argmax16(x[:, _ALU_LO:_ALU_LO + 16])
    a_hi = _argmax16(x[:, _ALU_HI:_ALU_HI + 16])
    b_lo = _argmax16(x[:, _AX_CARRY_LO:_AX_CARRY_LO + 16])
    b_hi = _argmax16(x[:, _AX_CARRY_HI:_AX_CARRY_HI + 16])

    byte_a = a_lo + (a_hi << 4)
    byte_b = b_lo + (b_hi << 4)
    result = (byte_a * byte_b) & 255
    res_lo = result & 15
    res_hi = result >> 4

    two_m = jnp.where(mask, jnp.float32(2.0), jnp.float32(0.0))
    add = jnp.where(lane == res_lo + _OUTPUT_LO, two_m, 0.0) + jnp.where(
        lane == res_hi + _OUTPUT_HI, two_m, 0.0)
    o_ref[...] = x + add


def kernel(x_bd):
    b, s, d = x_bd.shape
    rows = b * s
    x2 = x_bd.reshape(rows, d)
    block_rows = 1024
    out = pl.pallas_call(
        _body,
        grid=(rows // block_rows,),
        in_specs=[pl.BlockSpec((block_rows, d), lambda i: (i, 0))],
        out_specs=pl.BlockSpec((block_rows, d), lambda i: (i, 0)),
        out_shape=jax.ShapeDtypeStruct((rows, d), x_bd.dtype),
        compiler_params=pltpu.CompilerParams(
            dimension_semantics=("arbitrary",)),
    )(x2)
    return out.reshape(b, s, d)


# speedup vs baseline: 6.1060x; 2.0624x over previous
import jax
import jax.numpy as jnp
from jax.experimental import pallas as pl
from jax.experimental.pallas import tpu as pltpu

_MARK_AX = 0
_OP_MUL = 1
_ALU_LO = 16
_ALU_HI = 32
_AX_CARRY_LO = 48
_AX_CARRY_HI = 64
_OUTPUT_LO = 80
_OUTPUT_HI = 96
_DIM = 128


def _idx_from_mask(b_col):
    e = (jax.lax.bitcast_convert_type(b_col, jnp.int32) >> 23) - 127
    return 15 - e


def _body(x_ref, o_ref):
    x = x_ref[...]
    r = x.shape[0]
    lane = jax.lax.broadcasted_iota(jnp.int32, (r, _DIM), 1)

    v = x
    for s in (1, 2, 4, 8):
        left = pltpu.roll(v, _DIM - s, 1)
        right = pltpu.roll(v, s, 1)
        partner = jnp.where((lane & s) == 0, left, right)
        v = jnp.maximum(v, partner)

    onehot = jnp.where(x == v, jnp.float32(1.0), jnp.float32(0.0))

    ji = jax.lax.broadcasted_iota(jnp.int32, (_DIM, _DIM), 0)
    ci = jax.lax.broadcasted_iota(jnp.int32, (_DIM, _DIM), 1)
    w = jnp.where((ji >> 4) == (ci >> 4),
                  (jnp.int32(1) << (15 - (ji & 15))).astype(jnp.float32),
                  jnp.float32(0.0))
    bmask = jax.lax.dot_general(onehot, w, (((1,), (0,)), ((), ())),
                                preferred_element_type=jnp.float32)

    a_lo = _idx_from_mask(bmask[:, _ALU_LO:_ALU_LO + 1])
    a_hi = _idx_from_mask(bmask[:, _ALU_HI:_ALU_HI + 1])
    b_lo = _idx_from_mask(bmask[:, _AX_CARRY_LO:_AX_CARRY_LO + 1])
    b_hi = _idx_from_mask(bmask[:, _AX_CARRY_HI:_AX_CARRY_HI + 1])

    byte_a = a_lo + (a_hi << 4)
    byte_b = b_lo + (b_hi << 4)
    result = (byte_a * byte_b) & 255
    res_lo = result & 15
    res_hi = result >> 4

    mask = (x[:, _MARK_AX:_MARK_AX + 1] >= 0.5) & (
        x[:, _OP_MUL:_OP_MUL + 1] >= 0.5)
    two_m = jnp.where(mask, jnp.float32(2.0), jnp.float32(0.0))
    add = jnp.where(lane == res_lo + _OUTPUT_LO, two_m, 0.0) + jnp.where(
        lane == res_hi + _OUTPUT_HI, two_m, 0.0)
    o_ref[...] = x + add


def kernel(x_bd):
    b, s, d = x_bd.shape
    rows = b * s
    x2 = x_bd.reshape(rows, d)
    block_rows = 1024
    out = pl.pallas_call(
        _body,
        grid=(rows // block_rows,),
        in_specs=[pl.BlockSpec((block_rows, d), lambda i: (i, 0))],
        out_specs=pl.BlockSpec((block_rows, d), lambda i: (i, 0)),
        out_shape=jax.ShapeDtypeStruct((rows, d), x_bd.dtype),
        compiler_params=pltpu.CompilerParams(
            dimension_semantics=("arbitrary",)),
    )(x2)
    return out.reshape(b, s, d)


# speedup vs baseline: 8.6254x; 1.4126x over previous
import numpy as np
import jax
import jax.numpy as jnp
from jax.experimental import pallas as pl
from jax.experimental.pallas import tpu as pltpu

_DIM = 128


def _build_mats():
    bc = np.zeros((_DIM, _DIM), np.float32)
    w2 = np.zeros((_DIM, _DIM), np.float32)
    w3 = np.zeros((_DIM, _DIM), np.float32)
    for c in range(_DIM):
        bc[(c // 16) * 16, c] = 1.0
    for j in range(16, 80):
        g, k = j // 16, j % 16
        for c in range(g * 16, g * 16 + 16):
            w2[j, c] = float(1 << (15 - k))
    w3[16, 80:96] = 1.0
    w3[32, 80:96] = 16.0
    w3[48, 96:112] = 1.0
    w3[64, 96:112] = 16.0
    w3[0, 80:112] = 256.0
    w3[1, 80:112] = 256.0
    return jnp.asarray(bc), jnp.asarray(w2), jnp.asarray(w3)


def _mm(a, b):
    return jax.lax.dot_general(a, b, (((1,), (0,)), ((), ())),
                               preferred_element_type=jnp.float32)


def _body(x_ref, bc_ref, w2_ref, w3_ref, o_ref):
    x = x_ref[...]
    r = x.shape[0]
    lane = jax.lax.broadcasted_iota(jnp.int32, (r, _DIM), 1)

    ge01 = jnp.where(x >= 0.5, jnp.float32(1.0), jnp.float32(0.0))

    v = x
    for s in (1, 2, 4, 8):
        v = jnp.maximum(v, pltpu.roll(v, _DIM - s, 1))
    wsel = jnp.where((lane & 15) == 0, v, jnp.float32(0.0))
    gmax = _mm(wsel, bc_ref[...])

    onehot = jnp.where(x == gmax, jnp.float32(1.0), jnp.float32(0.0))
    z = jnp.where(lane < 2, ge01, onehot)
    bmask = _mm(z, w2_ref[...])

    idx = 142 - (jax.lax.bitcast_convert_type(bmask, jnp.int32) >> 23)
    in3 = jnp.where(lane < 2, ge01, idx.astype(jnp.float32))
    o3 = _mm(in3, w3_ref[...])

    vi = o3.astype(jnp.int32)
    s2 = (vi >> 8) == 2
    b8 = vi & 255
    vb = pltpu.roll(vi, _DIM - 16, 1)
    prod = (b8 * (vb & 255)) & 255
    prodr = pltpu.roll(prod, 16, 1)

    cond_lo = ((lane >> 4) == 5) & s2 & ((lane & 15) == (prod & 15))
    cond_hi = ((lane >> 4) == 6) & s2 & ((lane & 15) == ((prodr >> 4) & 15))
    add = jnp.where(cond_lo | cond_hi, jnp.float32(2.0), jnp.float32(0.0))
    o_ref[...] = x + add


def kernel(x_bd):
    b, s, d = x_bd.shape
    rows = b * s
    x2 = x_bd.reshape(rows, d)
    bc, w2, w3 = _build_mats()
    block_rows = 1024
    const_spec = pl.BlockSpec((d, d), lambda i: (0, 0))
    out = pl.pallas_call(
        _body,
        grid=(rows // block_rows,),
        in_specs=[pl.BlockSpec((block_rows, d), lambda i: (i, 0)),
                  const_spec, const_spec, const_spec],
        out_specs=pl.BlockSpec((block_rows, d), lambda i: (i, 0)),
        out_shape=jax.ShapeDtypeStruct((rows, d), x_bd.dtype),
        compiler_params=pltpu.CompilerParams(
            dimension_semantics=("arbitrary",)),
    )(x2, bc, w2, w3)
    return out.reshape(b, s, d)
